# baseline (speedup 1.0000x reference)
import functools

import jax
import jax.numpy as jnp
from jax import lax
from jax.experimental import pallas as pl
from jax.experimental.pallas import tpu as pltpu
from jax.experimental.pallas import tpu_sc as plsc

IN_FEATURES = 512
NUM_CODEBOOKS = 2
NUM_CODEWORDS = 320
NCOL = NUM_CODEBOOKS * NUM_CODEWORDS
CODEWORD_DIM = 128
ROWS = 4 * 1024
BLK = 512
GRID = ROWS // BLK

NC, NS = 2, 16
NW = NC * NS


def _tc_body(x_ref, w_ref, b_ref, ids_ref, perp_ref, counts_ref):
    pid = pl.program_id(0)

    @pl.when(pid == 0)
    def _init():
        counts_ref[...] = jnp.zeros_like(counts_ref)

    logits = (
        jnp.dot(x_ref[0], w_ref[...], preferred_element_type=jnp.float32)
        + b_ref[...]
    )
    iota = lax.broadcasted_iota(jnp.int32, (BLK, NCOL), 1)
    big = jnp.int32(2**30)
    neg = jnp.float32(-1e30)
    idxs = []
    for n in range(NUM_CODEBOOKS):
        in_cb = (iota >= n * NUM_CODEWORDS) & (iota < (n + 1) * NUM_CODEWORDS)
        lm = jnp.where(in_cb, logits, neg)
        m = jnp.max(lm, axis=1, keepdims=True)
        cand = jnp.where(lm == m, iota, big)
        idxs.append(jnp.min(cand, axis=1, keepdims=True))
    onehot = ((iota == idxs[0]) | (iota == idxs[1])).astype(jnp.float32)
    counts_ref[...] += lax.dot_general(
        jnp.ones((1, BLK), jnp.float32),
        onehot,
        (((1,), (0,)), ((), ())),
        preferred_element_type=jnp.float32,
    )
    iota_row = lax.broadcasted_iota(jnp.int32, (1, NCOL), 1).astype(jnp.float32)
    sel = jnp.concatenate(
        [
            jnp.where(iota_row < NUM_CODEWORDS, iota_row, 0.0),
            jnp.where(iota_row >= NUM_CODEWORDS, iota_row, 0.0),
        ],
        axis=0,
    )
    idx_lane = lax.dot_general(
        sel,
        onehot,
        (((1,), (1,)), ((), ())),
        preferred_element_type=jnp.float32,
        precision=lax.Precision.HIGHEST,
    )
    pieces = [
        idx_lane[n : n + 1, 128 * k : 128 * (k + 1)]
        for n in range(NUM_CODEBOOKS)
        for k in range(BLK // 128)
    ]
    ids_ref[...] = jnp.concatenate(pieces, axis=0).astype(jnp.int32)

    @pl.when(pid == GRID - 1)
    def _fin():
        p = counts_ref[...] * (1.0 / ROWS)
        plogp = p * jnp.log(p + 1e-7)
        e0 = jnp.sum(plogp[:, :NUM_CODEWORDS])
        e1 = jnp.sum(plogp[:, NUM_CODEWORDS:])
        perp_ref[...] = jnp.broadcast_to(jnp.exp(-e0) + jnp.exp(-e1), (1, 1))


def _tc_stage(x, w, b_row):
    return pl.pallas_call(
        _tc_body,
        grid=(GRID,),
        in_specs=[
            pl.BlockSpec((1, BLK, IN_FEATURES), lambda i: (i // 2, i % 2, 0)),
            pl.BlockSpec((IN_FEATURES, NCOL), lambda i: (0, 0)),
            pl.BlockSpec((1, NCOL), lambda i: (0, 0)),
        ],
        out_specs=[
            pl.BlockSpec((8, 128), lambda i: (i, 0)),
            pl.BlockSpec((1, 1), lambda i: (0, 0)),
        ],
        out_shape=[
            jax.ShapeDtypeStruct((8 * GRID, 128), jnp.int32),
            jax.ShapeDtypeStruct((1, 1), jnp.float32),
        ],
        scratch_shapes=[pltpu.VMEM((1, NCOL), jnp.float32)],
    )(x, w, b_row)


@functools.lru_cache(maxsize=1)
def _make_sc_gather():
    @functools.partial(
        pl.kernel,
        mesh=plsc.VectorSubcoreMesh(core_axis_name="c", subcore_axis_name="s"),
        out_type=jax.ShapeDtypeStruct((NUM_CODEBOOKS, ROWS, CODEWORD_DIM), jnp.float32),
        scratch_types=[
            pltpu.VMEM((2, 128), jnp.int32),
            pltpu.VMEM((256, CODEWORD_DIM), jnp.float32),
            pltpu.SemaphoreType.DMA,
        ],
    )
    def _sc_gather(table_hbm, idx_hbm, out_hbm, idx_v, rows_v, sem):
        wid = lax.axis_index("s") * NC + lax.axis_index("c")
        g = wid // 4
        q = wid % 4
        n = q // 2
        h = q % 2
        pltpu.sync_copy(idx_hbm.at[pl.ds(8 * g + 4 * n + 2 * h, 2)], idx_v)
        copies = []
        for j in range(2):
            copies.append(
                pltpu.async_copy(
                    table_hbm.at[idx_v.at[j]],
                    rows_v.at[pl.ds(j * 128, 128)],
                    sem,
                )
            )
        for c in copies:
            c.wait()
        pltpu.sync_copy(
            rows_v,
            out_hbm.at[n].at[pl.ds(512 * g + 256 * h, 256)],
        )

    return _sc_gather


def kernel(x, codebooks, W, b):
    bsz, nf, _ = x.shape
    ids, perp = _tc_stage(x, W, b.reshape(1, NCOL))
    table = codebooks.reshape(NCOL, CODEWORD_DIM)
    halves = _make_sc_gather()(table, ids)
    rows = jnp.concatenate([halves[0], halves[1]], axis=-1)
    quantized = rows.reshape(bsz, nf, NUM_CODEBOOKS * CODEWORD_DIM)
    return quantized, perp.reshape(())


# speedup vs baseline: 1.3458x; 1.3458x over previous
import functools

import jax
import jax.numpy as jnp
from jax import lax
from jax.experimental import pallas as pl
from jax.experimental.pallas import tpu as pltpu
from jax.experimental.pallas import tpu_sc as plsc

IN_FEATURES = 512
NUM_CODEBOOKS = 2
NUM_CODEWORDS = 320
NCOL = NUM_CODEBOOKS * NUM_CODEWORDS
CODEWORD_DIM = 128
ROWS = 4 * 1024
BLK = 512
GRID = ROWS // BLK

NC, NS = 2, 16
NW = NC * NS


def _tc_body(x_ref, w_ref, b_ref, ids_ref, perp_ref, counts_ref):
    pid = pl.program_id(0)

    @pl.when(pid == 0)
    def _init():
        counts_ref[...] = jnp.zeros_like(counts_ref)

    logits = (
        jnp.dot(x_ref[0], w_ref[...], preferred_element_type=jnp.float32)
        + b_ref[...]
    )
    iota = lax.broadcasted_iota(jnp.int32, (BLK, NCOL), 1)
    big = jnp.int32(2**30)
    neg = jnp.float32(-1e30)
    idxs = []
    for n in range(NUM_CODEBOOKS):
        in_cb = (iota >= n * NUM_CODEWORDS) & (iota < (n + 1) * NUM_CODEWORDS)
        lm = jnp.where(in_cb, logits, neg)
        m = jnp.max(lm, axis=1, keepdims=True)
        cand = jnp.where(lm == m, iota, big)
        idxs.append(jnp.min(cand, axis=1, keepdims=True))
    onehot = ((iota == idxs[0]) | (iota == idxs[1])).astype(jnp.float32)
    counts_ref[...] += lax.dot_general(
        jnp.ones((1, BLK), jnp.float32),
        onehot,
        (((1,), (0,)), ((), ())),
        preferred_element_type=jnp.float32,
    )
    idx_lane = jnp.concatenate(idxs, axis=1).T
    pieces = [
        idx_lane[n : n + 1, 128 * k : 128 * (k + 1)]
        for n in range(NUM_CODEBOOKS)
        for k in range(BLK // 128)
    ]
    ids_ref[...] = jnp.concatenate(pieces, axis=0)

    @pl.when(pid == GRID - 1)
    def _fin():
        p = counts_ref[...] * (1.0 / ROWS)
        plogp = p * jnp.log(p + 1e-7)
        e0 = jnp.sum(plogp[:, :NUM_CODEWORDS])
        e1 = jnp.sum(plogp[:, NUM_CODEWORDS:])
        perp_ref[...] = jnp.broadcast_to(jnp.exp(-e0) + jnp.exp(-e1), (1, 1))


def _tc_stage(x, w, b_row):
    return pl.pallas_call(
        _tc_body,
        grid=(GRID,),
        in_specs=[
            pl.BlockSpec((1, BLK, IN_FEATURES), lambda i: (i // 2, i % 2, 0)),
            pl.BlockSpec((IN_FEATURES, NCOL), lambda i: (0, 0)),
            pl.BlockSpec((1, NCOL), lambda i: (0, 0)),
        ],
        out_specs=[
            pl.BlockSpec((8, 128), lambda i: (i, 0)),
            pl.BlockSpec((1, 1), lambda i: (0, 0)),
        ],
        out_shape=[
            jax.ShapeDtypeStruct((8 * GRID, 128), jnp.int32),
            jax.ShapeDtypeStruct((1, 1), jnp.float32),
        ],
        scratch_shapes=[pltpu.VMEM((1, NCOL), jnp.float32)],
    )(x, w, b_row)


@functools.lru_cache(maxsize=1)
def _make_sc_gather():
    @functools.partial(
        pl.kernel,
        mesh=plsc.VectorSubcoreMesh(core_axis_name="c", subcore_axis_name="s"),
        out_type=jax.ShapeDtypeStruct((ROWS, NUM_CODEBOOKS * CODEWORD_DIM), jnp.float32),
        scratch_types=[
            pltpu.VMEM((2, 128), jnp.int32),
            pltpu.VMEM((256, CODEWORD_DIM), jnp.float32),
            pltpu.SemaphoreType.DMA,
        ],
    )
    def _sc_gather(table_hbm, idx_hbm, out_hbm, idx_v, rows_v, sem):
        wid = lax.axis_index("s") * NC + lax.axis_index("c")
        g = wid // 4
        q = wid % 4
        n = q // 2
        h = q % 2
        pltpu.sync_copy(idx_hbm.at[pl.ds(8 * g + 4 * n + 2 * h, 2)], idx_v)
        copies = []
        for j in range(2):
            copies.append(
                pltpu.async_copy(
                    table_hbm.at[idx_v.at[j]],
                    rows_v.at[pl.ds(j * 128, 128)],
                    sem,
                )
            )
        for c in copies:
            c.wait()
        pltpu.sync_copy(
            rows_v,
            out_hbm.at[pl.ds(512 * g + 256 * h, 256), pl.ds(128 * n, 128)],
        )

    return _sc_gather


def kernel(x, codebooks, W, b):
    bsz, nf, _ = x.shape
    ids, perp = _tc_stage(x, W, b.reshape(1, NCOL))
    table = codebooks.reshape(NCOL, CODEWORD_DIM)
    rows = _make_sc_gather()(table, ids)
    quantized = rows.reshape(bsz, nf, NUM_CODEBOOKS * CODEWORD_DIM)
    return quantized, perp.reshape(())
